# baseline (speedup 1.0000x reference)
import functools

import jax
import jax.numpy as jnp
from jax import lax
from jax.experimental import pallas as pl
from jax.experimental.pallas import tpu as pltpu
from jax.experimental.pallas import tpu_sc as plsc

N = 10000
E = 320000
F = 128
NC = 2
NS = 16
NW = NC * NS
CH = 128
NCH = -(-((E + NW * CH - 1) // (NW * CH)) // 8) * 8
EPW = NCH * CH
EP = NW * EPW
NTRASH = 112
NPAD = N + NTRASH
RPT = NPAD // NS

_mesh = plsc.VectorSubcoreMesh(core_axis_name="c", subcore_axis_name="s")


def _flat_worker_id():
    return lax.axis_index("s") * NC + lax.axis_index("c")


@functools.partial(
    pl.kernel,
    out_type=jax.ShapeDtypeStruct((NC, NPAD, F), jnp.float32),
    mesh=_mesh,
    scratch_types=[
        pltpu.VMEM_SHARED((NPAD, F), jnp.float32),
        pltpu.VMEM((NCH, 128), jnp.int32),
        pltpu.VMEM((CH, F), jnp.float32),
        pltpu.SemaphoreType.DMA,
    ],
)
def _sc_degree(rowd_hbm, zeros_hbm, ones_hbm, out_hbm, acc, idx_v, upd_v, sem):
    c = lax.axis_index("c")
    s = lax.axis_index("s")
    w = _flat_worker_id()
    pltpu.sync_copy(zeros_hbm.at[pl.ds(s * RPT, RPT)], acc.at[pl.ds(s * RPT, RPT)])
    pltpu.sync_copy(ones_hbm, upd_v)
    pltpu.sync_copy(rowd_hbm.at[pl.ds(w * NCH, NCH)], idx_v)
    plsc.subcore_barrier()

    for j in range(8):
        pltpu.async_copy(upd_v, acc.at[idx_v.at[j]], sem, add=True)

    @pl.loop(8, NCH)
    def _(j):
        pltpu.make_async_copy(upd_v, acc.at[idx_v.at[0]], sem).wait()
        pltpu.async_copy(upd_v, acc.at[idx_v.at[j]], sem, add=True)

    @pl.loop(0, 8)
    def _(j):
        pltpu.make_async_copy(upd_v, acc.at[idx_v.at[0]], sem).wait()

    plsc.subcore_barrier()
    pltpu.sync_copy(acc.at[pl.ds(s * RPT, RPT)], out_hbm.at[c, pl.ds(s * RPT, RPT)])


HNCH = NCH // 2


@functools.partial(
    pl.kernel,
    out_type=jax.ShapeDtypeStruct((NC, NPAD, F), jnp.float32),
    mesh=_mesh,
    scratch_types=[
        pltpu.VMEM_SHARED((NPAD, F), jnp.float32),
        pltpu.VMEM((HNCH, 128), jnp.int32),
        pltpu.VMEM((HNCH, 128), jnp.int32),
        pltpu.VMEM((CH, F), jnp.float32),
        pltpu.VMEM((CH, F), jnp.float32),
        pltpu.SemaphoreType.DMA,
        pltpu.SemaphoreType.DMA,
    ],
)
def _sc_prop(u_hbm, rowg_hbm, cols_hbm, zeros_hbm, out_hbm,
             acc, gidx_v, sidx_v, buf0, buf1, sem0, sem1):
    c = lax.axis_index("c")
    s = lax.axis_index("s")
    w = _flat_worker_id()
    pltpu.sync_copy(zeros_hbm.at[pl.ds(s * RPT, RPT)], acc.at[pl.ds(s * RPT, RPT)])
    plsc.subcore_barrier()

    for h in range(2):
        base = w * NCH + h * HNCH
        pltpu.sync_copy(rowg_hbm.at[pl.ds(base, HNCH)], gidx_v)
        pltpu.sync_copy(cols_hbm.at[pl.ds(base, HNCH)], sidx_v)

        pltpu.async_copy(u_hbm.at[gidx_v.at[0]], buf0, sem0)
        pltpu.async_copy(u_hbm.at[gidx_v.at[1]], buf1, sem1)

        @pl.loop(0, HNCH - 2, step=2)
        def _(j):
            pltpu.make_async_copy(u_hbm.at[gidx_v.at[j]], buf0, sem0).wait()
            pltpu.sync_copy(buf0, acc.at[sidx_v.at[j]], add=True)
            pltpu.async_copy(u_hbm.at[gidx_v.at[j + 2]], buf0, sem0)

            pltpu.make_async_copy(u_hbm.at[gidx_v.at[j]], buf1, sem1).wait()
            pltpu.sync_copy(buf1, acc.at[sidx_v.at[j + 1]], add=True)
            pltpu.async_copy(u_hbm.at[gidx_v.at[j + 3]], buf1, sem1)

        pltpu.make_async_copy(u_hbm.at[gidx_v.at[0]], buf0, sem0).wait()
        pltpu.sync_copy(buf0, acc.at[sidx_v.at[HNCH - 2]], add=True)
        pltpu.make_async_copy(u_hbm.at[gidx_v.at[0]], buf1, sem1).wait()
        pltpu.sync_copy(buf1, acc.at[sidx_v.at[HNCH - 1]], add=True)

    plsc.subcore_barrier()
    pltpu.sync_copy(acc.at[pl.ds(s * RPT, RPT)], out_hbm.at[c, pl.ds(s * RPT, RPT)])


def _tc_acc0_body(f_ref, w_ref, acc0_ref):
    acc0_ref[...] = jnp.dot(f_ref[...], w_ref[0] - w_ref[2],
                            preferred_element_type=jnp.float32,
                            precision=lax.Precision.HIGHEST)


def _tc_prep_body(pp_ref, f_ref, dinv_ref, u0_ref):
    deg = pp_ref[0, :N, 0:1] + pp_ref[1, :N, 0:1]
    pos = deg > 0.0
    dinv = jnp.where(pos, lax.rsqrt(jnp.where(pos, deg, 1.0)), 0.0)
    dinv_ref[...] = dinv
    u0_ref[...] = dinv * f_ref[...]


def _tc_mid_body(p1_ref, dinv_ref, acc0_ref, w_ref, acc1_ref, u1_ref):
    g1 = p1_ref[0, :N, :] + p1_ref[1, :N, :]
    dinv = dinv_ref[...]
    tx1 = -dinv * g1
    acc1_ref[...] = acc0_ref[...] + jnp.dot(tx1, w_ref[1],
                                            preferred_element_type=jnp.float32,
                            precision=lax.Precision.HIGHEST)
    u1_ref[...] = dinv * tx1


def _tc_final_body(p2_ref, dinv_ref, acc1_ref, w_ref, cb_ref, fw_ref, fb_ref,
                   y_ref):
    g2 = p2_ref[0, :N, :] + p2_ref[1, :N, :]
    b = -dinv_ref[...] * jnp.dot(g2, 2.0 * w_ref[2],
                                 preferred_element_type=jnp.float32,
                            precision=lax.Precision.HIGHEST)
    out = acc1_ref[...] + b + cb_ref[...]
    h = jnp.tanh(out)
    y_ref[...] = jnp.dot(h, fw_ref[...], preferred_element_type=jnp.float32,
                            precision=lax.Precision.HIGHEST) \
        + fb_ref[...]


def kernel(features, adj, cheb_w, cheb_b, final_w, final_b):
    row = adj[0]
    col = adj[1]
    pad = EP - E
    ar = lax.iota(jnp.int32, pad)
    trash = N + (ar % NTRASH)
    rowg = jnp.concatenate([row, ar % N]).reshape(NW * NCH, 128)
    rowd = jnp.concatenate([row, trash]).reshape(NW * NCH, 128)
    cols = jnp.concatenate([col, trash]).reshape(NW * NCH, 128)

    zerosF = jnp.zeros((NPAD, F), jnp.float32)
    onesF = jnp.ones((CH, F), jnp.float32)

    deg_pp = _sc_degree(rowd, zerosF, onesF)

    acc0 = pl.pallas_call(
        _tc_acc0_body,
        out_shape=jax.ShapeDtypeStruct((N, F), jnp.float32),
    )(features, cheb_w)

    dinv, u0 = pl.pallas_call(
        _tc_prep_body,
        out_shape=[
            jax.ShapeDtypeStruct((N, 1), jnp.float32),
            jax.ShapeDtypeStruct((N, F), jnp.float32),
        ],
    )(deg_pp, features)

    p1 = _sc_prop(u0, rowg, cols, zerosF)

    acc1, u1 = pl.pallas_call(
        _tc_mid_body,
        out_shape=[
            jax.ShapeDtypeStruct((N, F), jnp.float32),
            jax.ShapeDtypeStruct((N, F), jnp.float32),
        ],
    )(p1, dinv, acc0, cheb_w)

    p2 = _sc_prop(u1, rowg, cols, zerosF)

    y = pl.pallas_call(
        _tc_final_body,
        out_shape=jax.ShapeDtypeStruct((N, 1), jnp.float32),
    )(p2, dinv, acc1, cheb_w, cheb_b.reshape(1, F), final_w,
      final_b.reshape(1, 1))

    return y.reshape(-1)


# speedup vs baseline: 1.0471x; 1.0471x over previous
import functools

import jax
import jax.numpy as jnp
from jax import lax
from jax.experimental import pallas as pl
from jax.experimental.pallas import tpu as pltpu
from jax.experimental.pallas import tpu_sc as plsc

N = 10000
E = 320000
F = 128
NC = 2
NS = 16
NW = NC * NS
CH = 128
NCH = -(-((E + NW * CH - 1) // (NW * CH)) // 8) * 8
EPW = NCH * CH
EP = NW * EPW
NTRASH = 112
NPAD = N + NTRASH
RPT = NPAD // NS

_mesh = plsc.VectorSubcoreMesh(core_axis_name="c", subcore_axis_name="s")


def _flat_worker_id():
    return lax.axis_index("s") * NC + lax.axis_index("c")


@functools.partial(
    pl.kernel,
    out_type=jax.ShapeDtypeStruct((NC, NPAD, F), jnp.float32),
    mesh=_mesh,
    scratch_types=[
        pltpu.VMEM_SHARED((NPAD, F), jnp.float32),
        pltpu.VMEM((NCH, 128), jnp.int32),
        pltpu.VMEM((CH, F), jnp.float32),
        pltpu.SemaphoreType.DMA,
    ],
)
def _sc_degree(rowd_hbm, zeros_hbm, ones_hbm, out_hbm, acc, idx_v, upd_v, sem):
    c = lax.axis_index("c")
    s = lax.axis_index("s")
    w = _flat_worker_id()
    pltpu.sync_copy(zeros_hbm.at[pl.ds(s * RPT, RPT)], acc.at[pl.ds(s * RPT, RPT)])
    pltpu.sync_copy(ones_hbm, upd_v)
    pltpu.sync_copy(rowd_hbm.at[pl.ds(w * NCH, NCH)], idx_v)
    plsc.subcore_barrier()

    for j in range(8):
        pltpu.async_copy(upd_v, acc.at[idx_v.at[j]], sem, add=True)

    @pl.loop(8, NCH)
    def _(j):
        pltpu.make_async_copy(upd_v, acc.at[idx_v.at[0]], sem).wait()
        pltpu.async_copy(upd_v, acc.at[idx_v.at[j]], sem, add=True)

    @pl.loop(0, 8)
    def _(j):
        pltpu.make_async_copy(upd_v, acc.at[idx_v.at[0]], sem).wait()

    plsc.subcore_barrier()
    pltpu.sync_copy(acc.at[pl.ds(s * RPT, RPT)], out_hbm.at[c, pl.ds(s * RPT, RPT)])


HNCH = NCH // 2


@functools.partial(
    pl.kernel,
    out_type=jax.ShapeDtypeStruct((NC, NPAD, F), jnp.float32),
    mesh=_mesh,
    scratch_types=[
        pltpu.VMEM_SHARED((NPAD, F), jnp.float32),
        pltpu.VMEM((HNCH, 128), jnp.int32),
        pltpu.VMEM((HNCH, 128), jnp.int32),
        pltpu.VMEM((CH, F), jnp.float32),
        pltpu.VMEM((CH, F), jnp.float32),
        pltpu.SemaphoreType.DMA,
        pltpu.SemaphoreType.DMA,
    ],
)
def _sc_prop(u_hbm, rowg_hbm, cols_hbm, zeros_hbm, out_hbm,
             acc, gidx_v, sidx_v, buf0, buf1, sem0, sem1):
    c = lax.axis_index("c")
    s = lax.axis_index("s")
    w = _flat_worker_id()
    pltpu.sync_copy(zeros_hbm.at[pl.ds(s * RPT, RPT)], acc.at[pl.ds(s * RPT, RPT)])
    plsc.subcore_barrier()

    for h in range(2):
        base = w * NCH + h * HNCH
        pltpu.sync_copy(rowg_hbm.at[pl.ds(base, HNCH)], gidx_v)
        pltpu.sync_copy(cols_hbm.at[pl.ds(base, HNCH)], sidx_v)

        pltpu.async_copy(u_hbm.at[gidx_v.at[0]], buf0, sem0)
        pltpu.async_copy(u_hbm.at[gidx_v.at[1]], buf1, sem1)

        @pl.loop(0, HNCH - 2, step=2)
        def _(j):
            pltpu.make_async_copy(u_hbm.at[gidx_v.at[j]], buf0, sem0).wait()
            pltpu.sync_copy(buf0, acc.at[sidx_v.at[j]], add=True)
            pltpu.async_copy(u_hbm.at[gidx_v.at[j + 2]], buf0, sem0)

            pltpu.make_async_copy(u_hbm.at[gidx_v.at[j]], buf1, sem1).wait()
            pltpu.sync_copy(buf1, acc.at[sidx_v.at[j + 1]], add=True)
            pltpu.async_copy(u_hbm.at[gidx_v.at[j + 3]], buf1, sem1)

        pltpu.make_async_copy(u_hbm.at[gidx_v.at[0]], buf0, sem0).wait()
        pltpu.sync_copy(buf0, acc.at[sidx_v.at[HNCH - 2]], add=True)
        pltpu.make_async_copy(u_hbm.at[gidx_v.at[0]], buf1, sem1).wait()
        pltpu.sync_copy(buf1, acc.at[sidx_v.at[HNCH - 1]], add=True)

    plsc.subcore_barrier()
    pltpu.sync_copy(acc.at[pl.ds(s * RPT, RPT)], out_hbm.at[c, pl.ds(s * RPT, RPT)])


def _tc_acc0_body(f_ref, w_ref, acc0_ref):
    acc0_ref[...] = jnp.dot(f_ref[...], w_ref[0] - w_ref[2],
                            preferred_element_type=jnp.float32)


def _tc_prep_body(pp_ref, f_ref, dinv_ref, u0_ref):
    deg = pp_ref[0, :N, 0:1] + pp_ref[1, :N, 0:1]
    pos = deg > 0.0
    dinv = jnp.where(pos, lax.rsqrt(jnp.where(pos, deg, 1.0)), 0.0)
    dinv_ref[...] = dinv
    u0_ref[...] = dinv * f_ref[...]


def _tc_mid_body(p1_ref, dinv_ref, acc0_ref, w_ref, acc1_ref, u1_ref):
    g1 = p1_ref[0, :N, :] + p1_ref[1, :N, :]
    dinv = dinv_ref[...]
    tx1 = -dinv * g1
    acc1_ref[...] = acc0_ref[...] + jnp.dot(tx1, w_ref[1],
                                            preferred_element_type=jnp.float32)
    u1_ref[...] = dinv * tx1


def _tc_final_body(p2_ref, dinv_ref, acc1_ref, w_ref, cb_ref, fw_ref, fb_ref,
                   y_ref):
    g2 = p2_ref[0, :N, :] + p2_ref[1, :N, :]
    b = -dinv_ref[...] * jnp.dot(g2, 2.0 * w_ref[2],
                                 preferred_element_type=jnp.float32)
    out = acc1_ref[...] + b + cb_ref[...]
    h = jnp.tanh(out)
    y_ref[...] = jnp.dot(h, fw_ref[...], preferred_element_type=jnp.float32) \
        + fb_ref[...]


def kernel(features, adj, cheb_w, cheb_b, final_w, final_b):
    row = adj[0]
    col = adj[1]
    pad = EP - E
    ar = lax.iota(jnp.int32, pad)
    trash = N + (ar % NTRASH)
    rowg = jnp.concatenate([row, ar % N]).reshape(NW * NCH, 128)
    rowd = jnp.concatenate([row, trash]).reshape(NW * NCH, 128)
    cols = jnp.concatenate([col, trash]).reshape(NW * NCH, 128)

    zerosF = jnp.zeros((NPAD, F), jnp.float32)
    onesF = jnp.ones((CH, F), jnp.float32)

    deg_pp = _sc_degree(rowd, zerosF, onesF)

    acc0 = pl.pallas_call(
        _tc_acc0_body,
        out_shape=jax.ShapeDtypeStruct((N, F), jnp.float32),
    )(features, cheb_w)

    dinv, u0 = pl.pallas_call(
        _tc_prep_body,
        out_shape=[
            jax.ShapeDtypeStruct((N, 1), jnp.float32),
            jax.ShapeDtypeStruct((N, F), jnp.float32),
        ],
    )(deg_pp, features)

    p1 = _sc_prop(u0, rowg, cols, zerosF)

    acc1, u1 = pl.pallas_call(
        _tc_mid_body,
        out_shape=[
            jax.ShapeDtypeStruct((N, F), jnp.float32),
            jax.ShapeDtypeStruct((N, F), jnp.float32),
        ],
    )(p1, dinv, acc0, cheb_w)

    p2 = _sc_prop(u1, rowg, cols, zerosF)

    y = pl.pallas_call(
        _tc_final_body,
        out_shape=jax.ShapeDtypeStruct((N, 1), jnp.float32),
    )(p2, dinv, acc1, cheb_w, cheb_b.reshape(1, F), final_w,
      final_b.reshape(1, 1))

    return y.reshape(-1)
